# Initial kernel scaffold; baseline (speedup 1.0000x reference)
import jax
import jax.numpy as jnp
from jax import lax
from jax.experimental import pallas as pl
from jax.experimental.pallas import tpu as pltpu
from jax.experimental.pallas import tpu_sc as plsc

VOCAB = 1000001
DIM = 64
N_TOTAL = 16384 * 50
NC, NS = 2, 16
NW = NC * NS
PER_W = N_TOTAL // NW
C = 128
NCHUNK = PER_W // C
NBUF = 8


def _body(idx_hbm, table_hbm, out_hbm, idx_v, rows_v, gsem, wsem):
    wid = lax.axis_index("s") * NC + lax.axis_index("c")
    row0 = wid * PER_W

    pltpu.sync_copy(idx_hbm.at[wid], idx_v)

    def gather(c, b):
        return pltpu.async_copy(
            table_hbm.at[idx_v.at[c]], rows_v.at[b], gsem.at[b])

    def wait_gather(b):
        pltpu.make_async_copy(
            table_hbm.at[pl.ds(0, C)], rows_v.at[b], gsem.at[b]).wait()

    def write(c, b):
        return pltpu.async_copy(
            rows_v.at[b], out_hbm.at[pl.ds(row0 + c * C, C)], wsem.at[b])

    def wait_write(b):
        pltpu.make_async_copy(
            rows_v.at[b], out_hbm.at[pl.ds(0, C)], wsem.at[b]).wait()

    for b in range(NBUF):
        gather(b, b)

    @pl.loop(0, NCHUNK - NBUF, step=NBUF)
    def _main(g):
        for b in range(NBUF):
            c = g + b
            wait_gather(b)
            write(c, b)
            wait_write(b)
            gather(c + NBUF, b)

    for b in range(NBUF):
        wait_gather(b)
        write(NCHUNK - NBUF + b, b)
    for b in range(NBUF):
        wait_write(b)


def kernel(data, ivectors_weight):
    idx = data.astype(jnp.int32).reshape(NW, NCHUNK, C)
    mesh = plsc.VectorSubcoreMesh(core_axis_name="c", subcore_axis_name="s")
    out = pl.kernel(
        _body,
        out_type=jax.ShapeDtypeStruct((N_TOTAL, DIM), jnp.float32),
        mesh=mesh,
        scratch_types=[
            pltpu.VMEM((NCHUNK, C), jnp.int32),
            pltpu.VMEM((NBUF, C, DIM), jnp.float32),
            pltpu.SemaphoreType.DMA((NBUF,)),
            pltpu.SemaphoreType.DMA((NBUF,)),
        ],
    )(idx, ivectors_weight)
    return out.reshape(data.shape[0], data.shape[1], DIM)


# speedup vs baseline: 1.8766x; 1.8766x over previous
import jax
import jax.numpy as jnp
from jax import lax
from jax.experimental import pallas as pl
from jax.experimental.pallas import tpu as pltpu
from jax.experimental.pallas import tpu_sc as plsc

VOCAB = 1000001
DIM = 64
N_TOTAL = 16384 * 50
NC, NS = 2, 16
NW = NC * NS
PER_W = N_TOTAL // NW
C = 128
NCHUNK = PER_W // C
NBUF = 8


def _body(idx_hbm, table_hbm, out_hbm, idx_v, rows_v, gsem, wsem):
    wid = lax.axis_index("s") * NC + lax.axis_index("c")
    row0 = wid * PER_W

    pltpu.sync_copy(idx_hbm.at[wid], idx_v)

    def gather(c, b):
        return pltpu.async_copy(
            table_hbm.at[idx_v.at[c]], rows_v.at[b], gsem.at[b])

    def wait_gather(b):
        pltpu.make_async_copy(
            table_hbm.at[pl.ds(0, C)], rows_v.at[b], gsem.at[b]).wait()

    def write(c, b):
        return pltpu.async_copy(
            rows_v.at[b], out_hbm.at[pl.ds(row0 + c * C, C)], wsem.at[b])

    def wait_write(b):
        pltpu.make_async_copy(
            rows_v.at[b], out_hbm.at[pl.ds(0, C)], wsem.at[b]).wait()

    for b in range(NBUF):
        gather(b, b)

    @pl.loop(0, NCHUNK - NBUF, step=NBUF)
    def _main(g):
        for b in range(NBUF):
            c = g + b
            wait_gather(b)
            write(c, b)
            wait_write(b)
            gather(c + NBUF, b)

    for b in range(NBUF):
        wait_gather(b)
        write(NCHUNK - NBUF + b, b)
    for b in range(NBUF):
        wait_write(b)


def kernel(data, ivectors_weight):
    idx = data.astype(jnp.int32).reshape(NW, NCHUNK, C)
    mesh = plsc.VectorSubcoreMesh(core_axis_name="c", subcore_axis_name="s")
    out = pl.kernel(
        _body,
        out_type=jax.ShapeDtypeStruct((N_TOTAL, DIM), jnp.float32),
        mesh=mesh,
        scratch_types=[
            pltpu.VMEM((NCHUNK, C), jnp.int32),
            pltpu.VMEM((NBUF, C, DIM), jnp.float32),
            pltpu.SemaphoreType.DMA((NBUF,)),
            pltpu.SemaphoreType.DMA((NBUF,)),
        ],
        compiler_params=pltpu.CompilerParams(use_tc_tiling_on_sc=False),
    )(idx, ivectors_weight)
    return out.reshape(data.shape[0], data.shape[1], DIM)
